# baseline (speedup 1.0000x reference)
import jax
import jax.numpy as jnp
from jax import lax
from jax.experimental import pallas as pl
from jax.experimental.pallas import tpu as pltpu
from jax.experimental.pallas import tpu_sc as plsc

B = 1024
W = 20
NNEG = 15
VOCAB = 100000
NDOCS = 50000
D = 128
T = 25
EPS = 1e-10

NPER = W + W * NNEG
SOUT = 384
NC = 2
NS = 16
NWORK = NC * NS
NB = B // NWORK
CHUNK = 80
NCHB = NPER // CHUNK
NCH = NB * NCHB
NBUF = 8
DK = D // 16

_SC_PARAMS = pltpu.CompilerParams(needs_layout_passes=False,
                                  use_tc_tiling_on_sc=False)
_MESH = dict(core_axis_name="c", subcore_axis_name="s",
             num_cores=NC, num_subcores=NS)


def _worker_base():
    return (lax.axis_index("s") * NC + lax.axis_index("c")) * NB


def _cv_body(doc_hbm, tgt_hbm, emb_hbm, emt_hbm, dtp_hbm, cvp_hbm,
             docv, tgtv, pv, ivv, etv, cvv, sem_a):
    base = _worker_base()

    pltpu.sync_copy(doc_hbm.at[pl.ds(base, NB)], docv)
    pltpu.sync_copy(tgt_hbm.at[pl.ds(base, NB)], tgtv)
    pltpu.sync_copy(emt_hbm, etv)

    pltpu.async_copy(dtp_hbm.at[docv], pv, sem_a).wait()
    pltpu.async_copy(emb_hbm.at[tgtv], ivv, sem_a).wait()

    def cv_one(b, _):
        p0 = pv[b, pl.ds(0, 16)]
        p1 = pv[b, pl.ds(16, 16)]
        for k in range(DK):
            acc = ivv[b, pl.ds(16 * k, 16)]
            for t in range(T):
                s = p0[t] if t < 16 else p1[t - 16]
                acc = acc + s * etv[t, pl.ds(16 * k, 16)]
            cvv[b, pl.ds(16 * k, 16)] = acc
        return 0

    lax.fori_loop(0, NB, cv_one, 0)

    pltpu.sync_copy(cvv, cvp_hbm.at[pl.ds(base, NB), pl.ds(0, D)])
    pltpu.sync_copy(pv, cvp_hbm.at[pl.ds(base, NB), pl.ds(D, 32)])


def _sc_cv(doc, tgt, emb_i, emb_t, dtp):
    f = pl.kernel(
        _cv_body,
        out_type=jax.ShapeDtypeStruct((B, D + 32), jnp.float32),
        mesh=plsc.VectorSubcoreMesh(**_MESH),
        scratch_types=[
            pltpu.VMEM((NB,), jnp.int32),
            pltpu.VMEM((NB,), jnp.int32),
            pltpu.VMEM((NB, 32), jnp.float32),
            pltpu.VMEM((NB, D), jnp.float32),
            pltpu.VMEM((T, D), jnp.float32),
            pltpu.VMEM((NB, D), jnp.float32),
            pltpu.SemaphoreType.DMA,
        ],
        compiler_params=_SC_PARAMS,
    )
    return f(doc, tgt, emb_i, emb_t, dtp)


def _score_body(idx_hbm, emb_hbm, cvp_hbm, s_hbm,
                idxv, cvv, rows0, rows1, rows2, rows3, rows4, rows5,
                rows6, rows7, sv,
                sem0, sem1, sem2, sem3, sem4, sem5, sem6, sem7):
    base = _worker_base()
    bufs = (rows0, rows1, rows2, rows3, rows4, rows5, rows6, rows7)
    sems = (sem0, sem1, sem2, sem3, sem4, sem5, sem6, sem7)

    pltpu.sync_copy(idx_hbm.at[pl.ds(base, NB)], idxv)
    pltpu.sync_copy(cvp_hbm.at[pl.ds(base, NB)], cvv)

    def start(c, buf, sem):
        b = c // NCHB
        cc = c - b * NCHB
        idx = idxv.at[b, pl.ds(pl.multiple_of(cc * CHUNK, 8), CHUNK)]
        pltpu.async_copy(emb_hbm.at[idx], buf, sem)

    lane = lax.iota(jnp.int32, 16)

    def compute(c, buf):
        b = c // NCHB
        cc = c - b * NCHB
        cv = [cvv[b, pl.ds(16 * k, 16)] for k in range(DK)]

        def group(g, _):
            r0 = g * 16
            sc = jnp.zeros((16,), jnp.float32)
            for j in range(16):
                row = buf.at[r0 + j]
                acc = row[pl.ds(0, 16)] * cv[0]
                for k in range(1, DK):
                    acc = acc + row[pl.ds(16 * k, 16)] * cv[k]
                sc = jnp.where(lane == j, jnp.sum(acc), sc)
            sv[b, pl.ds(pl.multiple_of(cc * CHUNK + r0, 8), 16)] = sc
            return 0

        lax.fori_loop(0, CHUNK // 16, group, 0)

    for j in range(NBUF):
        start(j, bufs[j], sems[j])

    def ring(i, _):
        c0 = NBUF * i
        for j in range(NBUF):
            pltpu.make_async_copy(
                emb_hbm.at[idxv.at[0, pl.ds(0, CHUNK)]], bufs[j],
                sems[j]).wait()
            compute(c0 + j, bufs[j])

            @pl.when(i < NCH // NBUF - 1)
            def _():
                start(c0 + j + NBUF, bufs[j], sems[j])
        return 0

    lax.fori_loop(0, NCH // NBUF, ring, 0)

    pltpu.sync_copy(sv, s_hbm.at[pl.ds(base, NB), pl.ds(0, NPER)])
    pltpu.sync_copy(cvv.at[pl.ds(0, NB), pl.ds(D, 32)],
                    s_hbm.at[pl.ds(base, NB), pl.ds(NPER, 32)])


def _sc_score(idx2d, emb_i, cvp):
    f = pl.kernel(
        _score_body,
        out_type=jax.ShapeDtypeStruct((B, SOUT), jnp.float32),
        mesh=plsc.VectorSubcoreMesh(**_MESH),
        scratch_types=[
            pltpu.VMEM((NB, NPER), jnp.int32),
            pltpu.VMEM((NB, D + 32), jnp.float32),
            pltpu.VMEM((CHUNK, D), jnp.float32),
            pltpu.VMEM((CHUNK, D), jnp.float32),
            pltpu.VMEM((CHUNK, D), jnp.float32),
            pltpu.VMEM((CHUNK, D), jnp.float32),
            pltpu.VMEM((CHUNK, D), jnp.float32),
            pltpu.VMEM((CHUNK, D), jnp.float32),
            pltpu.VMEM((CHUNK, D), jnp.float32),
            pltpu.VMEM((CHUNK, D), jnp.float32),
            pltpu.VMEM((NB, NPER), jnp.float32),
            pltpu.SemaphoreType.DMA,
            pltpu.SemaphoreType.DMA,
            pltpu.SemaphoreType.DMA,
            pltpu.SemaphoreType.DMA,
            pltpu.SemaphoreType.DMA,
            pltpu.SemaphoreType.DMA,
            pltpu.SemaphoreType.DMA,
            pltpu.SemaphoreType.DMA,
        ],
        compiler_params=_SC_PARAMS,
    )
    return f(idx2d, emb_i, cvp)


def _tc_body(s_ref, o_ref):
    s = s_ref[:, :NPER]
    w = s_ref[:, NPER + T]
    cols = lax.broadcasted_iota(jnp.int32, (B, NPER), 1)
    x = jnp.where(cols < W, s, -s)
    ll = jnp.log(jnp.clip(jax.nn.sigmoid(x), EPS, None))
    loss = -jnp.sum(ll, axis=1)
    wn = w * (B / jnp.sum(w))
    o_ref[...] = loss * wn


def _tc_loss(scores):
    return pl.pallas_call(
        _tc_body,
        out_shape=jax.ShapeDtypeStruct((B,), jnp.float32),
    )(scores)


@jax.jit
def kernel(doc, target, contexts, labels, nwords, emb_i, emb_t,
           doc_topic_table, docweights):
    del labels
    dtp = jnp.concatenate(
        [doc_topic_table, docweights[:, None],
         jnp.zeros((NDOCS, 32 - T - 1), jnp.float32)], axis=1)
    cvp = _sc_cv(doc.astype(jnp.int32), target.astype(jnp.int32),
                 emb_i, emb_t, dtp)
    idx2d = jnp.concatenate(
        [contexts.astype(jnp.int32), nwords.astype(jnp.int32)], axis=1)
    s = _sc_score(idx2d, emb_i, cvp)
    return _tc_loss(s)


# speedup vs baseline: 1.1957x; 1.1957x over previous
import jax
import jax.numpy as jnp
from jax import lax
from jax.experimental import pallas as pl
from jax.experimental.pallas import tpu as pltpu
from jax.experimental.pallas import tpu_sc as plsc

B = 1024
W = 20
NNEG = 15
VOCAB = 100000
NDOCS = 50000
D = 128
T = 25
EPS = 1e-10

NPER = W + W * NNEG
SOUT = 384
NC = 2
NS = 16
NWORK = NC * NS
NB = B // NWORK
CHUNK = 80
NCHB = NPER // CHUNK
NCH = NB * NCHB
NBUF = 4
DK = D // 16

_SC_PARAMS = pltpu.CompilerParams(needs_layout_passes=False,
                                  use_tc_tiling_on_sc=False)
_MESH = dict(core_axis_name="c", subcore_axis_name="s",
             num_cores=NC, num_subcores=NS)


def _worker_base():
    return (lax.axis_index("s") * NC + lax.axis_index("c")) * NB


def _cv_body(doc_hbm, tgt_hbm, emb_hbm, emt_hbm, dtp_hbm, cvp_hbm,
             docv, tgtv, pv, ivv, etv, cvv, sem_a):
    base = _worker_base()

    pltpu.sync_copy(doc_hbm.at[pl.ds(base, NB)], docv)
    pltpu.sync_copy(tgt_hbm.at[pl.ds(base, NB)], tgtv)
    pltpu.sync_copy(emt_hbm, etv)

    pltpu.async_copy(dtp_hbm.at[docv], pv, sem_a).wait()
    pltpu.async_copy(emb_hbm.at[tgtv], ivv, sem_a).wait()

    def cv_one(b, _):
        p0 = pv[b, pl.ds(0, 16)]
        p1 = pv[b, pl.ds(16, 16)]
        for k in range(DK):
            acc = ivv[b, pl.ds(16 * k, 16)]
            for t in range(T):
                s = p0[t] if t < 16 else p1[t - 16]
                acc = acc + s * etv[t, pl.ds(16 * k, 16)]
            cvv[b, pl.ds(16 * k, 16)] = acc
        return 0

    lax.fori_loop(0, NB, cv_one, 0)

    pltpu.sync_copy(cvv, cvp_hbm.at[pl.ds(base, NB), pl.ds(0, D)])
    pltpu.sync_copy(pv, cvp_hbm.at[pl.ds(base, NB), pl.ds(D, 32)])


def _sc_cv(doc, tgt, emb_i, emb_t, dtp):
    f = pl.kernel(
        _cv_body,
        out_type=jax.ShapeDtypeStruct((B, D + 32), jnp.float32),
        mesh=plsc.VectorSubcoreMesh(**_MESH),
        scratch_types=[
            pltpu.VMEM((NB,), jnp.int32),
            pltpu.VMEM((NB,), jnp.int32),
            pltpu.VMEM((NB, 32), jnp.float32),
            pltpu.VMEM((NB, D), jnp.float32),
            pltpu.VMEM((T, D), jnp.float32),
            pltpu.VMEM((NB, D), jnp.float32),
            pltpu.SemaphoreType.DMA,
        ],
        compiler_params=_SC_PARAMS,
    )
    return f(doc, tgt, emb_i, emb_t, dtp)


def _score_body(idx_hbm, emb_hbm, cvp_hbm, s_hbm,
                idxv, cvv, rows0, rows1, rows2, rows3, rows4, rows5,
                rows6, rows7, sv,
                sem0, sem1, sem2, sem3, sem4, sem5, sem6, sem7):
    base = _worker_base()
    bufs = (rows0, rows1, rows2, rows3, rows4, rows5, rows6, rows7)
    sems = (sem0, sem1, sem2, sem3, sem4, sem5, sem6, sem7)

    pltpu.sync_copy(idx_hbm.at[pl.ds(base, NB)], idxv)
    pltpu.sync_copy(cvp_hbm.at[pl.ds(base, NB)], cvv)

    def start(c, buf, sem):
        b = c // NCHB
        cc = c - b * NCHB
        idx = idxv.at[b, pl.ds(pl.multiple_of(cc * CHUNK, 8), CHUNK)]
        pltpu.async_copy(emb_hbm.at[idx], buf, sem)

    lane = lax.iota(jnp.int32, 16)

    def compute(c, buf):
        b = c // NCHB
        cc = c - b * NCHB
        cv = [cvv[b, pl.ds(16 * k, 16)] for k in range(DK)]

        def group(g, _):
            r0 = g * 16
            sc = jnp.zeros((16,), jnp.float32)
            for j in range(16):
                row = buf.at[r0 + j]
                acc = row[pl.ds(0, 16)] * cv[0]
                for k in range(1, DK):
                    acc = acc + row[pl.ds(16 * k, 16)] * cv[k]
                sc = jnp.where(lane == j, jnp.sum(acc), sc)
            sv[b, pl.ds(pl.multiple_of(cc * CHUNK + r0, 8), 16)] = sc
            return 0

        lax.fori_loop(0, CHUNK // 16, group, 0)

    for j in range(NBUF):
        start(j, bufs[j], sems[j])

    def ring(i, _):
        c0 = NBUF * i
        for j in range(NBUF):
            pltpu.make_async_copy(
                emb_hbm.at[idxv.at[0, pl.ds(0, CHUNK)]], bufs[j],
                sems[j]).wait()
            compute(c0 + j, bufs[j])

            @pl.when(i < NCH // NBUF - 1)
            def _():
                start(c0 + j + NBUF, bufs[j], sems[j])
        return 0

    lax.fori_loop(0, NCH // NBUF, ring, 0)

    pltpu.sync_copy(sv, s_hbm.at[pl.ds(base, NB), pl.ds(0, NPER)])
    pltpu.sync_copy(cvv.at[pl.ds(0, NB), pl.ds(D, 32)],
                    s_hbm.at[pl.ds(base, NB), pl.ds(NPER, 32)])


def _sc_score(idx2d, emb_i, cvp):
    f = pl.kernel(
        _score_body,
        out_type=jax.ShapeDtypeStruct((B, SOUT), jnp.float32),
        mesh=plsc.VectorSubcoreMesh(**_MESH),
        scratch_types=[
            pltpu.VMEM((NB, NPER), jnp.int32),
            pltpu.VMEM((NB, D + 32), jnp.float32),
            pltpu.VMEM((CHUNK, D), jnp.float32),
            pltpu.VMEM((CHUNK, D), jnp.float32),
            pltpu.VMEM((CHUNK, D), jnp.float32),
            pltpu.VMEM((CHUNK, D), jnp.float32),
            pltpu.VMEM((CHUNK, D), jnp.float32),
            pltpu.VMEM((CHUNK, D), jnp.float32),
            pltpu.VMEM((CHUNK, D), jnp.float32),
            pltpu.VMEM((CHUNK, D), jnp.float32),
            pltpu.VMEM((NB, NPER), jnp.float32),
            pltpu.SemaphoreType.DMA,
            pltpu.SemaphoreType.DMA,
            pltpu.SemaphoreType.DMA,
            pltpu.SemaphoreType.DMA,
            pltpu.SemaphoreType.DMA,
            pltpu.SemaphoreType.DMA,
            pltpu.SemaphoreType.DMA,
            pltpu.SemaphoreType.DMA,
        ],
        compiler_params=_SC_PARAMS,
    )
    return f(idx2d, emb_i, cvp)


def _tc_body(s_ref, o_ref):
    s = s_ref[:, :NPER]
    w = s_ref[:, NPER + T]
    cols = lax.broadcasted_iota(jnp.int32, (B, NPER), 1)
    x = jnp.where(cols < W, s, -s)
    ll = jnp.log(jnp.clip(jax.nn.sigmoid(x), EPS, None))
    loss = -jnp.sum(ll, axis=1)
    wn = w * (B / jnp.sum(w))
    o_ref[...] = loss * wn


def _tc_loss(scores):
    return pl.pallas_call(
        _tc_body,
        out_shape=jax.ShapeDtypeStruct((B,), jnp.float32),
    )(scores)


@jax.jit
def kernel(doc, target, contexts, labels, nwords, emb_i, emb_t,
           doc_topic_table, docweights):
    del labels
    dtp = jnp.concatenate(
        [doc_topic_table, docweights[:, None],
         jnp.zeros((NDOCS, 32 - T - 1), jnp.float32)], axis=1)
    cvp = _sc_cv(doc.astype(jnp.int32), target.astype(jnp.int32),
                 emb_i, emb_t, dtp)
    idx2d = jnp.concatenate(
        [contexts.astype(jnp.int32), nwords.astype(jnp.int32)], axis=1)
    s = _sc_score(idx2d, emb_i, cvp)
    return _tc_loss(s)
